# baseline (speedup 1.0000x reference)
import jax
import jax.numpy as jnp
from jax.experimental import pallas as pl
from jax.experimental.pallas import tpu as pltpu

H = 128
ROWS_PER_BLK = 1000


def _combine_body(v_ref, mask_ref, summed_ref, out_ref):
    out_ref[...] = v_ref[...] + mask_ref[...] * summed_ref[...]


def kernel(x, v, cross_edge_index, W_prog, b_prog, W_vox, b_vox, Wm1, bm1, Wm2, bm2, theta):
    tau = 1.0
    src = cross_edge_index[0]
    dst = cross_edge_index[1]
    x_selected = jnp.take(x, src, axis=0)
    v_selected = jnp.take(v, dst, axis=0)
    h = jax.nn.leaky_relu(v @ Wm1.T + bm1, negative_slope=0.01)
    mask = jax.nn.sigmoid(h @ Wm2.T + bm2)
    e = theta.T * jnp.tanh(x_selected @ W_prog.T + b_prog + v_selected @ W_vox.T + b_vox)
    e = e.sum(axis=1)
    u = jax.random.uniform(jax.random.fold_in(jax.random.key(0), 1), e.shape,
                           minval=1e-10, maxval=1.0, dtype=jnp.float32)
    gumbel_noise = -jnp.log(-jnp.log(u))
    y = jax.nn.softmax((e + gumbel_noise) / tau, axis=0)
    num_v = v.shape[0]
    seg_max = jax.ops.segment_max(y, dst, num_segments=num_v)
    eids = jnp.arange(y.shape[0], dtype=jnp.int32)
    cand = jnp.where(y == jnp.take(seg_max, dst, axis=0), eids, jnp.int32(y.shape[0]))
    y_max = jax.ops.segment_min(cand, dst, num_segments=num_v)
    y_hard = jnp.zeros_like(y).at[y_max].set(1.0, mode='drop')
    y_hard = y_hard - jax.lax.stop_gradient(y) + y
    attention_soft = y[:, None]
    attention_hard = y_hard[:, None]
    summed = jax.ops.segment_sum(x_selected * attention_soft, dst, num_segments=num_v)

    nblk = v.shape[0] // ROWS_PER_BLK
    v_out = pl.pallas_call(
        _combine_body,
        grid=(nblk,),
        in_specs=[
            pl.BlockSpec((ROWS_PER_BLK, H), lambda i: (i, 0)),
            pl.BlockSpec((ROWS_PER_BLK, 1), lambda i: (i, 0)),
            pl.BlockSpec((ROWS_PER_BLK, H), lambda i: (i, 0)),
        ],
        out_specs=pl.BlockSpec((ROWS_PER_BLK, H), lambda i: (i, 0)),
        out_shape=jax.ShapeDtypeStruct((num_v, H), jnp.float32),
    )(v, mask, summed)
    return (v_out, mask, attention_soft, attention_hard)


# speedup vs baseline: 1.9626x; 1.9626x over previous
import jax
import jax.numpy as jnp
from jax import lax
from jax.experimental import pallas as pl
from jax.experimental.pallas import tpu as pltpu
from jax.experimental.pallas import tpu_sc as plsc

H = 128
NROW = 10000
E = 320000
NC = 2
NS = 16
NW = NC * NS
EPW = E // NW
C = 80
NCHUNK = EPW // C
G = C // 16
C2 = 2000
NCH2 = EPW // C2
G2 = C2 // 16
NP = 10240
NPR = NP // 16
RPT = NP // NS
RPTR = RPT // 16
ROWS_PER_BLK = 1000
EBLK = E // 10
INT_MAX = jnp.int32(2147483647)

_SC_MESH = dict(
    mesh=plsc.VectorSubcoreMesh(core_axis_name="c", subcore_axis_name="s"),
    compiler_params=pltpu.CompilerParams(
        needs_layout_passes=False, use_tc_tiling_on_sc=False),
)


def _dense_body(x_ref, v_ref, Wp_ref, bp_ref, Wv_ref, Wm1_ref, bm1_ref,
                Wm2_ref, bm2_ref, xp_ref, vp_ref, mask_ref):
    dn = (((1,), (1,)), ((), ()))
    xb = x_ref[...]
    vb = v_ref[...]
    xp_ref[...] = lax.dot_general(xb, Wp_ref[...], dn) + bp_ref[...][None, :]
    vp_ref[...] = lax.dot_general(vb, Wv_ref[...], dn)
    hm = lax.dot_general(vb, Wm1_ref[...], dn) + bm1_ref[...][None, :]
    hm = jnp.where(hm >= 0, hm, 0.01 * hm)
    mm = jnp.sum(hm * Wm2_ref[...], axis=1, keepdims=True) + bm2_ref[0]
    mask_ref[...] = jax.nn.sigmoid(mm)


def _dense_precompute(x, v, W_prog, b_prog, W_vox, Wm1, bm1, Wm2, bm2):
    nblk = NROW // ROWS_PER_BLK
    return pl.pallas_call(
        _dense_body,
        grid=(nblk,),
        in_specs=[
            pl.BlockSpec((ROWS_PER_BLK, H), lambda i: (i, 0)),
            pl.BlockSpec((ROWS_PER_BLK, H), lambda i: (i, 0)),
            pl.BlockSpec((H, H), lambda i: (0, 0)),
            pl.BlockSpec((H,), lambda i: (0,)),
            pl.BlockSpec((H, H), lambda i: (0, 0)),
            pl.BlockSpec((H, H), lambda i: (0, 0)),
            pl.BlockSpec((H,), lambda i: (0,)),
            pl.BlockSpec((1, H), lambda i: (0, 0)),
            pl.BlockSpec(memory_space=pltpu.SMEM),
        ],
        out_specs=[
            pl.BlockSpec((ROWS_PER_BLK, H), lambda i: (i, 0)),
            pl.BlockSpec((ROWS_PER_BLK, H), lambda i: (i, 0)),
            pl.BlockSpec((ROWS_PER_BLK, 1), lambda i: (i, 0)),
        ],
        out_shape=[
            jax.ShapeDtypeStruct((NROW, H), jnp.float32),
            jax.ShapeDtypeStruct((NROW, H), jnp.float32),
            jax.ShapeDtypeStruct((NROW, 1), jnp.float32),
        ],
    )(x, v, W_prog, b_prog, W_vox, Wm1, bm1, Wm2, bm2)


def _tanh(z):
    az = jnp.abs(z)
    ez = jnp.exp(az * -2.0)
    t = (1.0 - ez) / (1.0 + ez)
    return jnp.where(z < 0.0, -t, t)


def _score_body(xp_hbm, vp_hbm, src_hbm, dst_hbm, th_hbm, bv_hbm, e_hbm,
                isA, idA, xrA, vrA, isB, idB, xrB, vrB,
                evec, thv, bvv, semA, semB):
    wid = lax.axis_index("s") * NC + lax.axis_index("c")
    base_w = wid * EPW
    pltpu.sync_copy(th_hbm, thv)
    pltpu.sync_copy(bv_hbm, bvv)
    iota = lax.iota(jnp.int32, 16)

    def stage(base, idx_s, idx_d, xrows, vrows, sem):
        pltpu.sync_copy(src_hbm.at[pl.ds(base, C)], idx_s)
        pltpu.sync_copy(dst_hbm.at[pl.ds(base, C)], idx_d)
        pltpu.async_copy(xp_hbm.at[idx_s], xrows, sem)
        pltpu.async_copy(vp_hbm.at[idx_d], vrows, sem)

    def wait2(xrows, vrows, sem):
        pltpu.make_async_copy(xp_hbm.at[pl.ds(0, C)], xrows, sem).wait()
        pltpu.make_async_copy(vp_hbm.at[pl.ds(0, C)], vrows, sem).wait()

    def compute(base, xrows, vrows):
        def h_body(hh, accs):
            th = thv[hh]
            bv = bvv[hh]
            hsplat = jnp.full((16,), hh, dtype=jnp.int32)
            out = []
            for g in range(G):
                ei = iota + (g * 16)
                xa = plsc.load_gather(xrows, [ei, hsplat])
                vb = plsc.load_gather(vrows, [ei, hsplat])
                z = (xa + vb) + bv
                out.append(accs[g] + th * _tanh(z))
            return tuple(out)

        accs0 = tuple(jnp.zeros((16,), jnp.float32) for _ in range(G))
        accs = lax.fori_loop(0, H, h_body, accs0)
        for g in range(G):
            evec[pl.ds(g * 16, 16)] = accs[g]
        pltpu.sync_copy(evec, e_hbm.at[pl.ds(base, C)])

    stage(base_w, isA, idA, xrA, vrA, semA)

    def pair(pi, carry):
        b0 = base_w + (2 * pi) * C
        stage(b0 + C, isB, idB, xrB, vrB, semB)
        wait2(xrA, vrA, semA)
        compute(b0, xrA, vrA)
        stage(b0 + 2 * C, isA, idA, xrA, vrA, semA)
        wait2(xrB, vrB, semB)
        compute(b0 + C, xrB, vrB)
        return carry

    lax.fori_loop(0, (NCHUNK - 1) // 2, pair, 0)
    wait2(xrA, vrA, semA)
    compute(base_w + (NCHUNK - 1) * C, xrA, vrA)


def _edge_scores(xp, vp, src, dst, theta_b, bvox_b):
    f = pl.kernel(
        _score_body,
        out_type=jax.ShapeDtypeStruct((E,), jnp.float32),
        scratch_types=[
            pltpu.VMEM((C,), jnp.int32),
            pltpu.VMEM((C,), jnp.int32),
            pltpu.VMEM((C, H), jnp.float32),
            pltpu.VMEM((C, H), jnp.float32),
            pltpu.VMEM((C,), jnp.int32),
            pltpu.VMEM((C,), jnp.int32),
            pltpu.VMEM((C, H), jnp.float32),
            pltpu.VMEM((C, H), jnp.float32),
            pltpu.VMEM((C,), jnp.float32),
            pltpu.VMEM((H, 16), jnp.float32),
            pltpu.VMEM((H, 16), jnp.float32),
            pltpu.SemaphoreType.DMA,
            pltpu.SemaphoreType.DMA,
        ],
        **_SC_MESH,
    )
    return f(xp, vp, src, dst, theta_b, bvox_b)


def _softmax_body(e_ref, g_ref, y_ref):
    s = e_ref[...] + g_ref[...]
    m = jnp.max(s)
    p = jnp.exp(s - m)
    y_ref[...] = p / jnp.sum(p)


def _softmax(e2d, g2d):
    return pl.pallas_call(
        _softmax_body,
        out_shape=jax.ShapeDtypeStruct(e2d.shape, jnp.float32),
    )(e2d, g2d)


def _dhi(d):
    return lax.shift_right_logical(d, 4)


def _dlo(d):
    return lax.bitwise_and(d, 15)


def _segmax_body(y_hbm, dst_hbm, out_hbm, yv, dv, tbl, mrg, tmp, shared, sem):
    cid = lax.axis_index("c")
    sid = lax.axis_index("s")
    wid = sid * NC + cid
    base_w = wid * EPW
    zero16 = jnp.zeros((16,), jnp.float32)

    def z_body(i, c):
        tbl[i] = zero16
        return c

    lax.fori_loop(0, NPR, z_body, 0)

    def chunk(ci, carry):
        base = base_w + ci * C2
        pltpu.sync_copy(y_hbm.at[pl.ds(base, C2)], yv)
        pltpu.sync_copy(dst_hbm.at[pl.ds(base, C2)], dv)

        def grp(g, carry2):
            d = dv[pl.ds(g * 16, 16)]
            yy = yv[pl.ds(g * 16, 16)]
            hi = _dhi(d)
            lo = _dlo(d)

            def cond(p):
                return jnp.any(p)

            def body(p):
                cur = plsc.load_gather(tbl, [hi, lo], mask=p)
                new = jnp.maximum(cur, yy)
                plsc.store_scatter(tbl, [hi, lo], new, mask=p)
                cur2 = plsc.load_gather(tbl, [hi, lo], mask=p)
                return jnp.logical_and(p, cur2 < yy)

            lax.while_loop(cond, body, jnp.ones((16,), jnp.bool_))
            return carry2

        lax.fori_loop(0, G2, grp, 0)
        return carry

    lax.fori_loop(0, NCH2, chunk, 0)

    pltpu.sync_copy(tbl, shared.at[sid])
    plsc.subcore_barrier()
    pltpu.sync_copy(shared.at[0, pl.ds(sid * RPTR, RPTR)], mrg)
    for k in range(1, NS):
        pltpu.sync_copy(shared.at[k, pl.ds(sid * RPTR, RPTR)], tmp)

        def mx(i, c):
            mrg[i] = jnp.maximum(mrg[i], tmp[i])
            return c

        lax.fori_loop(0, RPTR, mx, 0)
    pltpu.sync_copy(mrg, out_hbm.at[cid, pl.ds(sid * RPTR, RPTR)])


def _segment_max(y, dst):
    f = pl.kernel(
        _segmax_body,
        out_type=jax.ShapeDtypeStruct((NC, NPR, 16), jnp.float32),
        scratch_types=[
            pltpu.VMEM((C2,), jnp.float32),
            pltpu.VMEM((C2,), jnp.int32),
            pltpu.VMEM((NPR, 16), jnp.float32),
            pltpu.VMEM((RPTR, 16), jnp.float32),
            pltpu.VMEM((RPTR, 16), jnp.float32),
            pltpu.VMEM_SHARED((NS, NPR, 16), jnp.float32),
            pltpu.SemaphoreType.DMA,
        ],
        **_SC_MESH,
    )
    return f(y, dst)


def _argmin_body(y_hbm, dst_hbm, mx_hbm, out_hbm,
                 yv, dv, mtbl, ftmp, itbl, mrg, tmp, shared, sem):
    cid = lax.axis_index("c")
    sid = lax.axis_index("s")
    wid = sid * NC + cid
    base_w = wid * EPW
    iota = lax.iota(jnp.int32, 16)
    big = jnp.full((16,), INT_MAX, dtype=jnp.int32)

    pltpu.sync_copy(mx_hbm.at[0], mtbl)
    pltpu.sync_copy(mx_hbm.at[1], ftmp)

    def mrg_mx(i, c):
        mtbl[i] = jnp.maximum(mtbl[i], ftmp[i])
        return c

    lax.fori_loop(0, NPR, mrg_mx, 0)

    def z_body(i, c):
        itbl[i] = big
        return c

    lax.fori_loop(0, NPR, z_body, 0)

    def chunk(ci, carry):
        base = base_w + ci * C2
        pltpu.sync_copy(y_hbm.at[pl.ds(base, C2)], yv)
        pltpu.sync_copy(dst_hbm.at[pl.ds(base, C2)], dv)

        def grp(g, carry2):
            d = dv[pl.ds(g * 16, 16)]
            yy = yv[pl.ds(g * 16, 16)]
            eid = base + g * 16 + iota
            hi = _dhi(d)
            lo = _dlo(d)
            mx = plsc.load_gather(mtbl, [hi, lo])
            sel = yy == mx

            def cond(p):
                return jnp.any(p)

            def body(p):
                cur = plsc.load_gather(itbl, [hi, lo], mask=p)
                new = jnp.minimum(cur, eid)
                plsc.store_scatter(itbl, [hi, lo], new, mask=p)
                cur2 = plsc.load_gather(itbl, [hi, lo], mask=p)
                return jnp.logical_and(p, cur2 > eid)

            lax.while_loop(cond, body, sel)
            return carry2

        lax.fori_loop(0, G2, grp, 0)
        return carry

    lax.fori_loop(0, NCH2, chunk, 0)

    pltpu.sync_copy(itbl, shared.at[sid])
    plsc.subcore_barrier()
    pltpu.sync_copy(shared.at[0, pl.ds(sid * RPTR, RPTR)], mrg)
    for k in range(1, NS):
        pltpu.sync_copy(shared.at[k, pl.ds(sid * RPTR, RPTR)], tmp)

        def mn(i, c):
            mrg[i] = jnp.minimum(mrg[i], tmp[i])
            return c

        lax.fori_loop(0, RPTR, mn, 0)
    pltpu.sync_copy(mrg, out_hbm.at[cid, pl.ds(sid * RPTR, RPTR)])


def _segment_argmin(y, dst, segmax_p):
    f = pl.kernel(
        _argmin_body,
        out_type=jax.ShapeDtypeStruct((NC, NPR, 16), jnp.int32),
        scratch_types=[
            pltpu.VMEM((C2,), jnp.float32),
            pltpu.VMEM((C2,), jnp.int32),
            pltpu.VMEM((NPR, 16), jnp.float32),
            pltpu.VMEM((NPR, 16), jnp.float32),
            pltpu.VMEM((NPR, 16), jnp.int32),
            pltpu.VMEM((RPTR, 16), jnp.int32),
            pltpu.VMEM((RPTR, 16), jnp.int32),
            pltpu.VMEM_SHARED((NS, NPR, 16), jnp.int32),
            pltpu.SemaphoreType.DMA,
        ],
        **_SC_MESH,
    )
    return f(y, dst, segmax_p)


ZROWS = 64


HHALF = H // 2


def _sum_body(y2_hbm, src_hbm, dst_hbm, x_hbm, slo_hbm, shi_hbm,
              yvg, sv, dv, xrows, xhalf, acc, sem):
    cid = lax.axis_index("c")
    sid = lax.axis_index("s")
    wid = sid * NC + cid
    base_w = wid * EPW
    zero16 = jnp.zeros((16,), jnp.float32)

    for phase, out_hbm in ((0, slo_hbm), (1, shi_hbm)):
        hoff = phase * HHALF

        def zb(i, c):
            for j in range(HHALF // 16):
                xhalf[i, pl.ds(j * 16, 16)] = zero16
            return c

        lax.fori_loop(0, C, zb, 0)
        for b in range(RPT // C):
            pltpu.sync_copy(xhalf, acc.at[pl.ds(sid * RPT + b * C, C)])
        plsc.subcore_barrier()

        def chunk(ci, carry):
            base = base_w + ci * C
            pltpu.sync_copy(y2_hbm.at[pl.ds(base_w // 16 + ci * G, G)], yvg)
            pltpu.sync_copy(src_hbm.at[pl.ds(base, C)], sv)
            pltpu.sync_copy(dst_hbm.at[pl.ds(base, C)], dv)
            pltpu.async_copy(x_hbm.at[sv], xrows, sem).wait()

            def rw(r, c2):
                rhi = jnp.full((16,), lax.shift_right_logical(r, 4), jnp.int32)
                rlo = jnp.full((16,), lax.bitwise_and(r, 15), jnp.int32)
                ys = plsc.load_gather(yvg, [rhi, rlo])
                for j in range(HHALF // 16):
                    xhalf[r, pl.ds(j * 16, 16)] = (
                        xrows[r, pl.ds(hoff + j * 16, 16)] * ys)
                return c2

            lax.fori_loop(0, C, rw, 0)
            pltpu.sync_copy(xhalf, acc.at[dv], add=True)
            return carry

        lax.fori_loop(0, NCHUNK, chunk, 0)
        plsc.subcore_barrier()
        pltpu.sync_copy(acc.at[pl.ds(sid * RPT, RPT)],
                        out_hbm.at[cid, pl.ds(sid * RPT, RPT)])
        plsc.subcore_barrier()


def _weighted_sum(y, src, dst, x):
    f = pl.kernel(
        _sum_body,
        out_type=[
            jax.ShapeDtypeStruct((NC, NP, HHALF), jnp.float32),
            jax.ShapeDtypeStruct((NC, NP, HHALF), jnp.float32),
        ],
        scratch_types=[
            pltpu.VMEM((G, 16), jnp.float32),
            pltpu.VMEM((C,), jnp.int32),
            pltpu.VMEM((C,), jnp.int32),
            pltpu.VMEM((C, H), jnp.float32),
            pltpu.VMEM((C, HHALF), jnp.float32),
            pltpu.VMEM_SHARED((NP, HHALF), jnp.float32),
            pltpu.SemaphoreType.DMA,
        ],
        **_SC_MESH,
    )
    return f(y.reshape(E // 16, 16), src, dst, x)


def _yhard_body(dst_hbm, id_hbm, yh_hbm, dv, yh, idtbl, itmp, sem):
    cid = lax.axis_index("c")
    sid = lax.axis_index("s")
    wid = sid * NC + cid
    base_w = wid * EPW
    iota = lax.iota(jnp.int32, 16)

    pltpu.sync_copy(id_hbm.at[0], idtbl)
    pltpu.sync_copy(id_hbm.at[1], itmp)

    def mrg_mn(i, c):
        idtbl[i] = jnp.minimum(idtbl[i], itmp[i])
        return c

    lax.fori_loop(0, NPR, mrg_mn, 0)

    def chunk(ci, carry):
        base = base_w + ci * C2
        pltpu.sync_copy(dst_hbm.at[pl.ds(base, C2)], dv)

        def grp(g, c2):
            d = dv[pl.ds(g * 16, 16)]
            eid = base + g * 16 + iota
            idg = plsc.load_gather(idtbl, [_dhi(d), _dlo(d)])
            yh[pl.ds(g * 16, 16)] = jnp.where(idg == eid, 1.0, 0.0)
            return c2

        lax.fori_loop(0, G2, grp, 0)
        pltpu.sync_copy(yh, yh_hbm.at[pl.ds(base, C2)])
        return carry

    lax.fori_loop(0, NCH2, chunk, 0)


def _hard_onehot(dst, minid_p):
    f = pl.kernel(
        _yhard_body,
        out_type=jax.ShapeDtypeStruct((E,), jnp.float32),
        scratch_types=[
            pltpu.VMEM((C2,), jnp.int32),
            pltpu.VMEM((C2,), jnp.float32),
            pltpu.VMEM((NPR, 16), jnp.int32),
            pltpu.VMEM((NPR, 16), jnp.int32),
            pltpu.SemaphoreType.DMA,
        ],
        **_SC_MESH,
    )
    return f(dst, minid_p)


def _final_body(v_ref, mask_ref, s0lo_ref, s1lo_ref, s0hi_ref, s1hi_ref, vout_ref):
    summed = jnp.concatenate(
        [s0lo_ref[0] + s1lo_ref[0], s0hi_ref[0] + s1hi_ref[0]], axis=1)
    vout_ref[...] = v_ref[...] + mask_ref[...] * summed


def _ah_body(y_ref, yh_ref, ah_ref):
    yb = y_ref[...]
    ah_ref[...] = (yh_ref[...] - yb) + yb


def _final(v, mask, sum_lo, sum_hi, y2d, yh2d):
    v_out = pl.pallas_call(
        _final_body,
        grid=(10,),
        in_specs=[
            pl.BlockSpec((ROWS_PER_BLK, H), lambda i: (i, 0)),
            pl.BlockSpec((ROWS_PER_BLK, 1), lambda i: (i, 0)),
            pl.BlockSpec((1, ROWS_PER_BLK, HHALF), lambda i: (0, i, 0)),
            pl.BlockSpec((1, ROWS_PER_BLK, HHALF), lambda i: (1, i, 0)),
            pl.BlockSpec((1, ROWS_PER_BLK, HHALF), lambda i: (0, i, 0)),
            pl.BlockSpec((1, ROWS_PER_BLK, HHALF), lambda i: (1, i, 0)),
        ],
        out_specs=pl.BlockSpec((ROWS_PER_BLK, H), lambda i: (i, 0)),
        out_shape=jax.ShapeDtypeStruct((NROW, H), jnp.float32),
    )(v, mask, sum_lo, sum_lo, sum_hi, sum_hi)
    ah2d = pl.pallas_call(
        _ah_body,
        out_shape=jax.ShapeDtypeStruct((E // H, H), jnp.float32),
    )(y2d, yh2d)
    return v_out, ah2d


def kernel(x, v, cross_edge_index, W_prog, b_prog, W_vox, b_vox, Wm1, bm1, Wm2, bm2, theta):
    src = cross_edge_index[0]
    dst = cross_edge_index[1]

    xp, vp, mask = _dense_precompute(
        x, v, W_prog, b_prog, W_vox, Wm1, bm1, Wm2, bm2)
    theta_b = jnp.broadcast_to(theta.reshape(H, 1), (H, 16))
    bvox_b = jnp.broadcast_to(b_vox.reshape(H, 1), (H, 16))
    e = _edge_scores(xp, vp, src, dst, theta_b, bvox_b)

    u = jax.random.uniform(jax.random.fold_in(jax.random.key(0), 1), (E,),
                           minval=1e-10, maxval=1.0, dtype=jnp.float32)
    gumbel_noise = -jnp.log(-jnp.log(u))
    y2d = _softmax(e.reshape(E // H, H), gumbel_noise.reshape(E // H, H))
    y = y2d.reshape(E)

    segmax_p = _segment_max(y, dst)
    minid_p = _segment_argmin(y, dst, segmax_p)
    sum_lo, sum_hi = _weighted_sum(y, src, dst, x)
    yh = _hard_onehot(dst, minid_p)
    v_out, ah2d = _final(v, mask, sum_lo, sum_hi, y2d, yh.reshape(E // H, H))

    return (v_out, mask, y[:, None], ah2d.reshape(E, 1))


# speedup vs baseline: 3.4409x; 1.7533x over previous
import jax
import jax.numpy as jnp
from jax import lax
from jax.experimental import pallas as pl
from jax.experimental.pallas import tpu as pltpu
from jax.experimental.pallas import tpu_sc as plsc

H = 128
NROW = 10000
E = 320000
NC = 2
NS = 16
NW = NC * NS
EPW = E // NW
C = 80
NCHUNK = EPW // C
G = C // 16
C2 = 2000
NCH2 = EPW // C2
G2 = C2 // 16
NP = 10240
NPR = NP // 16
RPT = NP // NS
RPTR = RPT // 16
ROWS_PER_BLK = 1000
EBLK = E // 10
INT_MAX = jnp.int32(2147483647)

_SC_MESH = dict(
    mesh=plsc.VectorSubcoreMesh(core_axis_name="c", subcore_axis_name="s"),
    compiler_params=pltpu.CompilerParams(
        needs_layout_passes=False, use_tc_tiling_on_sc=False),
)


def _dense_body(x_ref, v_ref, Wp_ref, bp_ref, Wv_ref, Wm1_ref, bm1_ref,
                Wm2_ref, bm2_ref, xp_ref, vp_ref, mask_ref):
    dn = (((1,), (1,)), ((), ()))
    xb = x_ref[...]
    vb = v_ref[...]
    xp_ref[...] = lax.dot_general(xb, Wp_ref[...], dn) + bp_ref[...][None, :]
    vp_ref[...] = lax.dot_general(vb, Wv_ref[...], dn)
    hm = lax.dot_general(vb, Wm1_ref[...], dn) + bm1_ref[...][None, :]
    hm = jnp.where(hm >= 0, hm, 0.01 * hm)
    mm = jnp.sum(hm * Wm2_ref[...], axis=1, keepdims=True) + bm2_ref[0]
    mask_ref[...] = jax.nn.sigmoid(mm)


def _dense_precompute(x, v, W_prog, b_prog, W_vox, Wm1, bm1, Wm2, bm2):
    nblk = NROW // ROWS_PER_BLK
    return pl.pallas_call(
        _dense_body,
        grid=(nblk,),
        in_specs=[
            pl.BlockSpec((ROWS_PER_BLK, H), lambda i: (i, 0)),
            pl.BlockSpec((ROWS_PER_BLK, H), lambda i: (i, 0)),
            pl.BlockSpec((H, H), lambda i: (0, 0)),
            pl.BlockSpec((H,), lambda i: (0,)),
            pl.BlockSpec((H, H), lambda i: (0, 0)),
            pl.BlockSpec((H, H), lambda i: (0, 0)),
            pl.BlockSpec((H,), lambda i: (0,)),
            pl.BlockSpec((1, H), lambda i: (0, 0)),
            pl.BlockSpec(memory_space=pltpu.SMEM),
        ],
        out_specs=[
            pl.BlockSpec((ROWS_PER_BLK, H), lambda i: (i, 0)),
            pl.BlockSpec((ROWS_PER_BLK, H), lambda i: (i, 0)),
            pl.BlockSpec((ROWS_PER_BLK, 1), lambda i: (i, 0)),
        ],
        out_shape=[
            jax.ShapeDtypeStruct((NROW, H), jnp.float32),
            jax.ShapeDtypeStruct((NROW, H), jnp.float32),
            jax.ShapeDtypeStruct((NROW, 1), jnp.float32),
        ],
    )(x, v, W_prog, b_prog, W_vox, Wm1, bm1, Wm2, bm2)


def _tanh(z):
    az = jnp.abs(z)
    ez = jnp.exp(az * -2.0)
    t = (1.0 - ez) / (1.0 + ez)
    return jnp.where(z < 0.0, -t, t)


def _score_body(xp_hbm, vp_hbm, src_hbm, dst_hbm, th_hbm, bv_hbm, e_hbm,
                isA, idA, xrA, vrA, isB, idB, xrB, vrB,
                evec, thv, bvv, semA, semB):
    wid = lax.axis_index("s") * NC + lax.axis_index("c")
    base_w = wid * EPW
    pltpu.sync_copy(th_hbm, thv)
    pltpu.sync_copy(bv_hbm, bvv)
    iota = lax.iota(jnp.int32, 16)

    def stage(base, idx_s, idx_d, xrows, vrows, sem):
        pltpu.sync_copy(src_hbm.at[pl.ds(base, C)], idx_s)
        pltpu.sync_copy(dst_hbm.at[pl.ds(base, C)], idx_d)
        pltpu.async_copy(xp_hbm.at[idx_s], xrows, sem)
        pltpu.async_copy(vp_hbm.at[idx_d], vrows, sem)

    def wait2(xrows, vrows, sem):
        pltpu.make_async_copy(xp_hbm.at[pl.ds(0, C)], xrows, sem).wait()
        pltpu.make_async_copy(vp_hbm.at[pl.ds(0, C)], vrows, sem).wait()

    def compute(base, xrows, vrows):
        def h_body(hh, accs):
            th = thv[hh]
            bv = bvv[hh]
            hvec = jnp.bitwise_and(hh + iota, H - 1)
            out = []
            for g in range(G):
                ei = iota + (g * 16)
                xa = plsc.load_gather(xrows, [ei, hvec])
                vb = plsc.load_gather(vrows, [ei, hvec])
                z = (xa + vb) + bv
                out.append(accs[g] + th * _tanh(z))
            return tuple(out)

        accs0 = tuple(jnp.zeros((16,), jnp.float32) for _ in range(G))
        accs = lax.fori_loop(0, H, h_body, accs0)
        for g in range(G):
            evec[pl.ds(g * 16, 16)] = accs[g]
        pltpu.sync_copy(evec, e_hbm.at[pl.ds(base, C)])

    stage(base_w, isA, idA, xrA, vrA, semA)

    def pair(pi, carry):
        b0 = base_w + (2 * pi) * C
        stage(b0 + C, isB, idB, xrB, vrB, semB)
        wait2(xrA, vrA, semA)
        compute(b0, xrA, vrA)
        stage(b0 + 2 * C, isA, idA, xrA, vrA, semA)
        wait2(xrB, vrB, semB)
        compute(b0 + C, xrB, vrB)
        return carry

    lax.fori_loop(0, (NCHUNK - 1) // 2, pair, 0)
    wait2(xrA, vrA, semA)
    compute(base_w + (NCHUNK - 1) * C, xrA, vrA)


def _edge_scores(xp, vp, src, dst, theta_b, bvox_b):
    f = pl.kernel(
        _score_body,
        out_type=jax.ShapeDtypeStruct((E,), jnp.float32),
        scratch_types=[
            pltpu.VMEM((C,), jnp.int32),
            pltpu.VMEM((C,), jnp.int32),
            pltpu.VMEM((C, H), jnp.float32),
            pltpu.VMEM((C, H), jnp.float32),
            pltpu.VMEM((C,), jnp.int32),
            pltpu.VMEM((C,), jnp.int32),
            pltpu.VMEM((C, H), jnp.float32),
            pltpu.VMEM((C, H), jnp.float32),
            pltpu.VMEM((C,), jnp.float32),
            pltpu.VMEM((H, 16), jnp.float32),
            pltpu.VMEM((H, 16), jnp.float32),
            pltpu.SemaphoreType.DMA,
            pltpu.SemaphoreType.DMA,
        ],
        **_SC_MESH,
    )
    return f(xp, vp, src, dst, theta_b, bvox_b)


def _softmax_body(e_ref, g_ref, y_ref):
    s = e_ref[...] + g_ref[...]
    m = jnp.max(s)
    p = jnp.exp(s - m)
    y_ref[...] = p / jnp.sum(p)


def _softmax(e2d, g2d):
    return pl.pallas_call(
        _softmax_body,
        out_shape=jax.ShapeDtypeStruct(e2d.shape, jnp.float32),
    )(e2d, g2d)


def _dhi(d):
    return lax.shift_right_logical(d, 4)


def _dlo(d):
    return lax.bitwise_and(d, 15)


def _segmax_body(y_hbm, dst_hbm, out_hbm, yv, dv, tbl, mrg, tmp, shared, sem):
    cid = lax.axis_index("c")
    sid = lax.axis_index("s")
    wid = sid * NC + cid
    base_w = wid * EPW
    zero16 = jnp.zeros((16,), jnp.float32)

    def z_body(i, c):
        tbl[i] = zero16
        return c

    lax.fori_loop(0, NPR, z_body, 0)

    def chunk(ci, carry):
        base = base_w + ci * C2
        pltpu.sync_copy(y_hbm.at[pl.ds(base, C2)], yv)
        pltpu.sync_copy(dst_hbm.at[pl.ds(base, C2)], dv)

        def grp(g, carry2):
            d = dv[pl.ds(g * 16, 16)]
            yy = yv[pl.ds(g * 16, 16)]
            hi = _dhi(d)
            lo = _dlo(d)

            def cond(p):
                return jnp.any(p)

            def body(p):
                cur = plsc.load_gather(tbl, [hi, lo], mask=p)
                new = jnp.maximum(cur, yy)
                plsc.store_scatter(tbl, [hi, lo], new, mask=p)
                cur2 = plsc.load_gather(tbl, [hi, lo], mask=p)
                return jnp.logical_and(p, cur2 < yy)

            lax.while_loop(cond, body, jnp.ones((16,), jnp.bool_))
            return carry2

        lax.fori_loop(0, G2, grp, 0)
        return carry

    lax.fori_loop(0, NCH2, chunk, 0)

    pltpu.sync_copy(tbl, shared.at[sid])
    plsc.subcore_barrier()
    pltpu.sync_copy(shared.at[0, pl.ds(sid * RPTR, RPTR)], mrg)
    for k in range(1, NS):
        pltpu.sync_copy(shared.at[k, pl.ds(sid * RPTR, RPTR)], tmp)

        def mx(i, c):
            mrg[i] = jnp.maximum(mrg[i], tmp[i])
            return c

        lax.fori_loop(0, RPTR, mx, 0)
    pltpu.sync_copy(mrg, out_hbm.at[cid, pl.ds(sid * RPTR, RPTR)])


def _segment_max(y, dst):
    f = pl.kernel(
        _segmax_body,
        out_type=jax.ShapeDtypeStruct((NC, NPR, 16), jnp.float32),
        scratch_types=[
            pltpu.VMEM((C2,), jnp.float32),
            pltpu.VMEM((C2,), jnp.int32),
            pltpu.VMEM((NPR, 16), jnp.float32),
            pltpu.VMEM((RPTR, 16), jnp.float32),
            pltpu.VMEM((RPTR, 16), jnp.float32),
            pltpu.VMEM_SHARED((NS, NPR, 16), jnp.float32),
            pltpu.SemaphoreType.DMA,
        ],
        **_SC_MESH,
    )
    return f(y, dst)


def _argmin_body(y_hbm, dst_hbm, mx_hbm, out_hbm,
                 yv, dv, mtbl, ftmp, itbl, mrg, tmp, shared, sem):
    cid = lax.axis_index("c")
    sid = lax.axis_index("s")
    wid = sid * NC + cid
    base_w = wid * EPW
    iota = lax.iota(jnp.int32, 16)
    big = jnp.full((16,), INT_MAX, dtype=jnp.int32)

    pltpu.sync_copy(mx_hbm.at[0], mtbl)
    pltpu.sync_copy(mx_hbm.at[1], ftmp)

    def mrg_mx(i, c):
        mtbl[i] = jnp.maximum(mtbl[i], ftmp[i])
        return c

    lax.fori_loop(0, NPR, mrg_mx, 0)

    def z_body(i, c):
        itbl[i] = big
        return c

    lax.fori_loop(0, NPR, z_body, 0)

    def chunk(ci, carry):
        base = base_w + ci * C2
        pltpu.sync_copy(y_hbm.at[pl.ds(base, C2)], yv)
        pltpu.sync_copy(dst_hbm.at[pl.ds(base, C2)], dv)

        def grp(g, carry2):
            d = dv[pl.ds(g * 16, 16)]
            yy = yv[pl.ds(g * 16, 16)]
            eid = base + g * 16 + iota
            hi = _dhi(d)
            lo = _dlo(d)
            mx = plsc.load_gather(mtbl, [hi, lo])
            sel = yy == mx

            def cond(p):
                return jnp.any(p)

            def body(p):
                cur = plsc.load_gather(itbl, [hi, lo], mask=p)
                new = jnp.minimum(cur, eid)
                plsc.store_scatter(itbl, [hi, lo], new, mask=p)
                cur2 = plsc.load_gather(itbl, [hi, lo], mask=p)
                return jnp.logical_and(p, cur2 > eid)

            lax.while_loop(cond, body, sel)
            return carry2

        lax.fori_loop(0, G2, grp, 0)
        return carry

    lax.fori_loop(0, NCH2, chunk, 0)

    pltpu.sync_copy(itbl, shared.at[sid])
    plsc.subcore_barrier()
    pltpu.sync_copy(shared.at[0, pl.ds(sid * RPTR, RPTR)], mrg)
    for k in range(1, NS):
        pltpu.sync_copy(shared.at[k, pl.ds(sid * RPTR, RPTR)], tmp)

        def mn(i, c):
            mrg[i] = jnp.minimum(mrg[i], tmp[i])
            return c

        lax.fori_loop(0, RPTR, mn, 0)
    pltpu.sync_copy(mrg, out_hbm.at[cid, pl.ds(sid * RPTR, RPTR)])


def _segment_argmin(y, dst, segmax_p):
    f = pl.kernel(
        _argmin_body,
        out_type=jax.ShapeDtypeStruct((NC, NPR, 16), jnp.int32),
        scratch_types=[
            pltpu.VMEM((C2,), jnp.float32),
            pltpu.VMEM((C2,), jnp.int32),
            pltpu.VMEM((NPR, 16), jnp.float32),
            pltpu.VMEM((NPR, 16), jnp.float32),
            pltpu.VMEM((NPR, 16), jnp.int32),
            pltpu.VMEM((RPTR, 16), jnp.int32),
            pltpu.VMEM((RPTR, 16), jnp.int32),
            pltpu.VMEM_SHARED((NS, NPR, 16), jnp.int32),
            pltpu.SemaphoreType.DMA,
        ],
        **_SC_MESH,
    )
    return f(y, dst, segmax_p)


ZROWS = 64


HHALF = H // 2


def _sum_body(y2_hbm, src_hbm, dst_hbm, x_hbm, slo_hbm, shi_hbm,
              yvg, sv, dv, xrows, xhalf, acc, sem):
    cid = lax.axis_index("c")
    sid = lax.axis_index("s")
    wid = sid * NC + cid
    base_w = wid * EPW
    zero16 = jnp.zeros((16,), jnp.float32)

    for phase, out_hbm in ((0, slo_hbm), (1, shi_hbm)):
        hoff = phase * HHALF

        def zb(i, c):
            for j in range(HHALF // 16):
                xhalf[i, pl.ds(j * 16, 16)] = zero16
            return c

        lax.fori_loop(0, C, zb, 0)
        for b in range(RPT // C):
            pltpu.sync_copy(xhalf, acc.at[pl.ds(sid * RPT + b * C, C)])
        plsc.subcore_barrier()

        def chunk(ci, carry):
            base = base_w + ci * C
            pltpu.sync_copy(y2_hbm.at[pl.ds(base_w // 16 + ci * G, G)], yvg)
            pltpu.sync_copy(src_hbm.at[pl.ds(base, C)], sv)
            pltpu.sync_copy(dst_hbm.at[pl.ds(base, C)], dv)
            pltpu.async_copy(x_hbm.at[sv], xrows, sem).wait()

            def rw(r, c2):
                rhi = jnp.full((16,), lax.shift_right_logical(r, 4), jnp.int32)
                rlo = jnp.full((16,), lax.bitwise_and(r, 15), jnp.int32)
                ys = plsc.load_gather(yvg, [rhi, rlo])
                for j in range(HHALF // 16):
                    xhalf[r, pl.ds(j * 16, 16)] = (
                        xrows[r, pl.ds(hoff + j * 16, 16)] * ys)
                return c2

            lax.fori_loop(0, C, rw, 0)
            pltpu.sync_copy(xhalf, acc.at[dv], add=True)
            return carry

        lax.fori_loop(0, NCHUNK, chunk, 0)
        plsc.subcore_barrier()
        pltpu.sync_copy(acc.at[pl.ds(sid * RPT, RPT)],
                        out_hbm.at[cid, pl.ds(sid * RPT, RPT)])
        plsc.subcore_barrier()


def _weighted_sum(y, src, dst, x):
    f = pl.kernel(
        _sum_body,
        out_type=[
            jax.ShapeDtypeStruct((NC, NP, HHALF), jnp.float32),
            jax.ShapeDtypeStruct((NC, NP, HHALF), jnp.float32),
        ],
        scratch_types=[
            pltpu.VMEM((G, 16), jnp.float32),
            pltpu.VMEM((C,), jnp.int32),
            pltpu.VMEM((C,), jnp.int32),
            pltpu.VMEM((C, H), jnp.float32),
            pltpu.VMEM((C, HHALF), jnp.float32),
            pltpu.VMEM_SHARED((NP, HHALF), jnp.float32),
            pltpu.SemaphoreType.DMA,
        ],
        **_SC_MESH,
    )
    return f(y.reshape(E // 16, 16), src, dst, x)


def _yhard_body(dst_hbm, id_hbm, yh_hbm, dv, yh, idtbl, itmp, sem):
    cid = lax.axis_index("c")
    sid = lax.axis_index("s")
    wid = sid * NC + cid
    base_w = wid * EPW
    iota = lax.iota(jnp.int32, 16)

    pltpu.sync_copy(id_hbm.at[0], idtbl)
    pltpu.sync_copy(id_hbm.at[1], itmp)

    def mrg_mn(i, c):
        idtbl[i] = jnp.minimum(idtbl[i], itmp[i])
        return c

    lax.fori_loop(0, NPR, mrg_mn, 0)

    def chunk(ci, carry):
        base = base_w + ci * C2
        pltpu.sync_copy(dst_hbm.at[pl.ds(base, C2)], dv)

        def grp(g, c2):
            d = dv[pl.ds(g * 16, 16)]
            eid = base + g * 16 + iota
            idg = plsc.load_gather(idtbl, [_dhi(d), _dlo(d)])
            yh[pl.ds(g * 16, 16)] = jnp.where(idg == eid, 1.0, 0.0)
            return c2

        lax.fori_loop(0, G2, grp, 0)
        pltpu.sync_copy(yh, yh_hbm.at[pl.ds(base, C2)])
        return carry

    lax.fori_loop(0, NCH2, chunk, 0)


def _hard_onehot(dst, minid_p):
    f = pl.kernel(
        _yhard_body,
        out_type=jax.ShapeDtypeStruct((E,), jnp.float32),
        scratch_types=[
            pltpu.VMEM((C2,), jnp.int32),
            pltpu.VMEM((C2,), jnp.float32),
            pltpu.VMEM((NPR, 16), jnp.int32),
            pltpu.VMEM((NPR, 16), jnp.int32),
            pltpu.SemaphoreType.DMA,
        ],
        **_SC_MESH,
    )
    return f(dst, minid_p)


def _final_body(v_ref, mask_ref, s0lo_ref, s1lo_ref, s0hi_ref, s1hi_ref, vout_ref):
    summed = jnp.concatenate(
        [s0lo_ref[0] + s1lo_ref[0], s0hi_ref[0] + s1hi_ref[0]], axis=1)
    vout_ref[...] = v_ref[...] + mask_ref[...] * summed


def _ah_body(y_ref, yh_ref, ah_ref):
    yb = y_ref[...]
    ah_ref[...] = (yh_ref[...] - yb) + yb


def _final(v, mask, sum_lo, sum_hi, y2d, yh2d):
    v_out = pl.pallas_call(
        _final_body,
        grid=(10,),
        in_specs=[
            pl.BlockSpec((ROWS_PER_BLK, H), lambda i: (i, 0)),
            pl.BlockSpec((ROWS_PER_BLK, 1), lambda i: (i, 0)),
            pl.BlockSpec((1, ROWS_PER_BLK, HHALF), lambda i: (0, i, 0)),
            pl.BlockSpec((1, ROWS_PER_BLK, HHALF), lambda i: (1, i, 0)),
            pl.BlockSpec((1, ROWS_PER_BLK, HHALF), lambda i: (0, i, 0)),
            pl.BlockSpec((1, ROWS_PER_BLK, HHALF), lambda i: (1, i, 0)),
        ],
        out_specs=pl.BlockSpec((ROWS_PER_BLK, H), lambda i: (i, 0)),
        out_shape=jax.ShapeDtypeStruct((NROW, H), jnp.float32),
    )(v, mask, sum_lo, sum_lo, sum_hi, sum_hi)
    ah2d = pl.pallas_call(
        _ah_body,
        out_shape=jax.ShapeDtypeStruct((E // H, H), jnp.float32),
    )(y2d, yh2d)
    return v_out, ah2d


def kernel(x, v, cross_edge_index, W_prog, b_prog, W_vox, b_vox, Wm1, bm1, Wm2, bm2, theta):
    src = cross_edge_index[0]
    dst = cross_edge_index[1]

    xp, vp, mask = _dense_precompute(
        x, v, W_prog, b_prog, W_vox, Wm1, bm1, Wm2, bm2)
    rot = (jnp.arange(H, dtype=jnp.int32)[:, None]
           + jnp.arange(16, dtype=jnp.int32)[None, :]) % H
    theta_b = theta.reshape(H)[rot]
    bvox_b = b_vox[rot]
    e = _edge_scores(xp, vp, src, dst, theta_b, bvox_b)

    u = jax.random.uniform(jax.random.fold_in(jax.random.key(0), 1), (E,),
                           minval=1e-10, maxval=1.0, dtype=jnp.float32)
    gumbel_noise = -jnp.log(-jnp.log(u))
    y2d = _softmax(e.reshape(E // H, H), gumbel_noise.reshape(E // H, H))
    y = y2d.reshape(E)

    segmax_p = _segment_max(y, dst)
    minid_p = _segment_argmin(y, dst, segmax_p)
    sum_lo, sum_hi = _weighted_sum(y, src, dst, x)
    yh = _hard_onehot(dst, minid_p)
    v_out, ah2d = _final(v, mask, sum_lo, sum_hi, y2d, yh.reshape(E // H, H))

    return (v_out, mask, y[:, None], ah2d.reshape(E, 1))


# speedup vs baseline: 4.0656x; 1.1815x over previous
import jax
import jax.numpy as jnp
from jax import lax
from jax.experimental import pallas as pl
from jax.experimental.pallas import tpu as pltpu
from jax.experimental.pallas import tpu_sc as plsc

H = 128
NROW = 10000
E = 320000
NC = 2
NS = 16
NW = NC * NS
EPW = E // NW
C = 80
NCHUNK = EPW // C
G = C // 16
C2 = 2000
NCH2 = EPW // C2
G2 = C2 // 16
NP = 10240
NPR = NP // 16
RPT = NP // NS
RPTR = RPT // 16
ROWS_PER_BLK = 1000
EBLK = E // 10
INT_MAX = jnp.int32(2147483647)

_SC_MESH = dict(
    mesh=plsc.VectorSubcoreMesh(core_axis_name="c", subcore_axis_name="s"),
    compiler_params=pltpu.CompilerParams(
        needs_layout_passes=False, use_tc_tiling_on_sc=False),
)


def _dense_body(x_ref, v_ref, Wp_ref, bp_ref, Wv_ref, Wm1_ref, bm1_ref,
                Wm2_ref, bm2_ref, xp_ref, vp_ref, mask_ref):
    dn = (((1,), (1,)), ((), ()))
    xb = x_ref[...]
    vb = v_ref[...]
    xp_ref[...] = lax.dot_general(xb, Wp_ref[...], dn) + bp_ref[...][None, :]
    vp_ref[...] = lax.dot_general(vb, Wv_ref[...], dn)
    hm = lax.dot_general(vb, Wm1_ref[...], dn) + bm1_ref[...][None, :]
    hm = jnp.where(hm >= 0, hm, 0.01 * hm)
    mm = jnp.sum(hm * Wm2_ref[...], axis=1, keepdims=True) + bm2_ref[0]
    mask_ref[...] = jax.nn.sigmoid(mm)


def _dense_precompute(x, v, W_prog, b_prog, W_vox, Wm1, bm1, Wm2, bm2):
    nblk = NROW // ROWS_PER_BLK
    return pl.pallas_call(
        _dense_body,
        grid=(nblk,),
        in_specs=[
            pl.BlockSpec((ROWS_PER_BLK, H), lambda i: (i, 0)),
            pl.BlockSpec((ROWS_PER_BLK, H), lambda i: (i, 0)),
            pl.BlockSpec((H, H), lambda i: (0, 0)),
            pl.BlockSpec((H,), lambda i: (0,)),
            pl.BlockSpec((H, H), lambda i: (0, 0)),
            pl.BlockSpec((H, H), lambda i: (0, 0)),
            pl.BlockSpec((H,), lambda i: (0,)),
            pl.BlockSpec((1, H), lambda i: (0, 0)),
            pl.BlockSpec(memory_space=pltpu.SMEM),
        ],
        out_specs=[
            pl.BlockSpec((ROWS_PER_BLK, H), lambda i: (i, 0)),
            pl.BlockSpec((ROWS_PER_BLK, H), lambda i: (i, 0)),
            pl.BlockSpec((ROWS_PER_BLK, 1), lambda i: (i, 0)),
        ],
        out_shape=[
            jax.ShapeDtypeStruct((NROW, H), jnp.float32),
            jax.ShapeDtypeStruct((NROW, H), jnp.float32),
            jax.ShapeDtypeStruct((NROW, 1), jnp.float32),
        ],
    )(x, v, W_prog, b_prog, W_vox, Wm1, bm1, Wm2, bm2)


def _tanh(z):
    az = jnp.abs(z)
    ez = jnp.exp(az * -2.0)
    t = (1.0 - ez) / (1.0 + ez)
    return jnp.where(z < 0.0, -t, t)


def _score_body(xp_hbm, vp_hbm, src_hbm, dst_hbm, th_hbm, bv_hbm, e_hbm,
                isA, idA, xrA, vrA, isB, idB, xrB, vrB,
                evec, thv, bvv, semA, semB):
    wid = lax.axis_index("s") * NC + lax.axis_index("c")
    base_w = wid * EPW
    pltpu.sync_copy(th_hbm, thv)
    pltpu.sync_copy(bv_hbm, bvv)
    iota = lax.iota(jnp.int32, 16)

    def stage(base, idx_s, idx_d, xrows, vrows, sem):
        pltpu.sync_copy(src_hbm.at[pl.ds(base, C)], idx_s)
        pltpu.sync_copy(dst_hbm.at[pl.ds(base, C)], idx_d)
        pltpu.async_copy(xp_hbm.at[idx_s], xrows, sem)
        pltpu.async_copy(vp_hbm.at[idx_d], vrows, sem)

    def wait2(xrows, vrows, sem):
        pltpu.make_async_copy(xp_hbm.at[pl.ds(0, C)], xrows, sem).wait()
        pltpu.make_async_copy(vp_hbm.at[pl.ds(0, C)], vrows, sem).wait()

    def compute(base, xrows, vrows):
        def h_body(hh, accs):
            th = thv[hh]
            bv = bvv[hh]
            hvec = jnp.bitwise_and(hh + iota, H - 1)
            out = []
            for g in range(G):
                ei = iota + (g * 16)
                xa = plsc.load_gather(xrows, [ei, hvec])
                vb = plsc.load_gather(vrows, [ei, hvec])
                z = (xa + vb) + bv
                out.append(accs[g] + th * _tanh(z))
            return tuple(out)

        accs0 = tuple(jnp.zeros((16,), jnp.float32) for _ in range(G))
        accs = lax.fori_loop(0, H, h_body, accs0)
        for g in range(G):
            evec[pl.ds(g * 16, 16)] = accs[g]
        pltpu.sync_copy(evec, e_hbm.at[pl.ds(base, C)])

    stage(base_w, isA, idA, xrA, vrA, semA)

    def pair(pi, carry):
        b0 = base_w + (2 * pi) * C
        stage(b0 + C, isB, idB, xrB, vrB, semB)
        wait2(xrA, vrA, semA)
        compute(b0, xrA, vrA)
        stage(b0 + 2 * C, isA, idA, xrA, vrA, semA)
        wait2(xrB, vrB, semB)
        compute(b0 + C, xrB, vrB)
        return carry

    lax.fori_loop(0, (NCHUNK - 1) // 2, pair, 0)
    wait2(xrA, vrA, semA)
    compute(base_w + (NCHUNK - 1) * C, xrA, vrA)


def _edge_scores(xp, vp, src, dst, theta_b, bvox_b):
    f = pl.kernel(
        _score_body,
        out_type=jax.ShapeDtypeStruct((E,), jnp.float32),
        scratch_types=[
            pltpu.VMEM((C,), jnp.int32),
            pltpu.VMEM((C,), jnp.int32),
            pltpu.VMEM((C, H), jnp.float32),
            pltpu.VMEM((C, H), jnp.float32),
            pltpu.VMEM((C,), jnp.int32),
            pltpu.VMEM((C,), jnp.int32),
            pltpu.VMEM((C, H), jnp.float32),
            pltpu.VMEM((C, H), jnp.float32),
            pltpu.VMEM((C,), jnp.float32),
            pltpu.VMEM((H, 16), jnp.float32),
            pltpu.VMEM((H, 16), jnp.float32),
            pltpu.SemaphoreType.DMA,
            pltpu.SemaphoreType.DMA,
        ],
        **_SC_MESH,
    )
    return f(xp, vp, src, dst, theta_b, bvox_b)


def _softmax_body(e_ref, g_ref, y_ref):
    s = e_ref[...] + g_ref[...]
    m = jnp.max(s)
    p = jnp.exp(s - m)
    y_ref[...] = p / jnp.sum(p)


def _softmax(e2d, g2d):
    return pl.pallas_call(
        _softmax_body,
        out_shape=jax.ShapeDtypeStruct(e2d.shape, jnp.float32),
    )(e2d, g2d)


def _dhi(d):
    return lax.shift_right_logical(d, 4)


def _dlo(d):
    return lax.bitwise_and(d, 15)


def _segmax_body(y_hbm, dst_hbm, out_hbm, yv, dv, tbl, mrg, tmp, shared, sem):
    cid = lax.axis_index("c")
    sid = lax.axis_index("s")
    wid = sid * NC + cid
    base_w = wid * EPW
    zero16 = jnp.zeros((16,), jnp.float32)

    def z_body(i, c):
        tbl[i] = zero16
        return c

    lax.fori_loop(0, NPR, z_body, 0)

    def chunk(ci, carry):
        base = base_w + ci * C2
        pltpu.sync_copy(y_hbm.at[pl.ds(base, C2)], yv)
        pltpu.sync_copy(dst_hbm.at[pl.ds(base, C2)], dv)

        def grp(g, carry2):
            d = dv[pl.ds(g * 16, 16)]
            yy = yv[pl.ds(g * 16, 16)]
            hi = _dhi(d)
            lo = _dlo(d)

            def cond(p):
                return jnp.any(p)

            def body(p):
                cur = plsc.load_gather(tbl, [hi, lo], mask=p)
                new = jnp.maximum(cur, yy)
                plsc.store_scatter(tbl, [hi, lo], new, mask=p)
                cur2 = plsc.load_gather(tbl, [hi, lo], mask=p)
                return jnp.logical_and(p, cur2 < yy)

            lax.while_loop(cond, body, jnp.ones((16,), jnp.bool_))
            return carry2

        lax.fori_loop(0, G2, grp, 0)
        return carry

    lax.fori_loop(0, NCH2, chunk, 0)

    pltpu.sync_copy(tbl, shared.at[sid])
    plsc.subcore_barrier()
    pltpu.sync_copy(shared.at[0, pl.ds(sid * RPTR, RPTR)], mrg)
    for k in range(1, NS):
        pltpu.sync_copy(shared.at[k, pl.ds(sid * RPTR, RPTR)], tmp)

        def mx(i, c):
            mrg[i] = jnp.maximum(mrg[i], tmp[i])
            return c

        lax.fori_loop(0, RPTR, mx, 0)
    pltpu.sync_copy(mrg, out_hbm.at[cid, pl.ds(sid * RPTR, RPTR)])


def _segment_max(y, dst):
    f = pl.kernel(
        _segmax_body,
        out_type=jax.ShapeDtypeStruct((NC, NPR, 16), jnp.float32),
        scratch_types=[
            pltpu.VMEM((C2,), jnp.float32),
            pltpu.VMEM((C2,), jnp.int32),
            pltpu.VMEM((NPR, 16), jnp.float32),
            pltpu.VMEM((RPTR, 16), jnp.float32),
            pltpu.VMEM((RPTR, 16), jnp.float32),
            pltpu.VMEM_SHARED((NS, NPR, 16), jnp.float32),
            pltpu.SemaphoreType.DMA,
        ],
        **_SC_MESH,
    )
    return f(y, dst)


def _argmin_body(y_hbm, dst_hbm, mx_hbm, out_hbm,
                 yv, dv, mtbl, ftmp, itbl, mrg, tmp, shared, sem):
    cid = lax.axis_index("c")
    sid = lax.axis_index("s")
    wid = sid * NC + cid
    base_w = wid * EPW
    iota = lax.iota(jnp.int32, 16)
    big = jnp.full((16,), INT_MAX, dtype=jnp.int32)

    pltpu.sync_copy(mx_hbm.at[0], mtbl)
    pltpu.sync_copy(mx_hbm.at[1], ftmp)

    def mrg_mx(i, c):
        mtbl[i] = jnp.maximum(mtbl[i], ftmp[i])
        return c

    lax.fori_loop(0, NPR, mrg_mx, 0)

    def z_body(i, c):
        itbl[i] = big
        return c

    lax.fori_loop(0, NPR, z_body, 0)

    def chunk(ci, carry):
        base = base_w + ci * C2
        pltpu.sync_copy(y_hbm.at[pl.ds(base, C2)], yv)
        pltpu.sync_copy(dst_hbm.at[pl.ds(base, C2)], dv)

        def grp(g, carry2):
            d = dv[pl.ds(g * 16, 16)]
            yy = yv[pl.ds(g * 16, 16)]
            eid = base + g * 16 + iota
            hi = _dhi(d)
            lo = _dlo(d)
            mx = plsc.load_gather(mtbl, [hi, lo])
            sel = yy == mx

            def cond(p):
                return jnp.any(p)

            def body(p):
                cur = plsc.load_gather(itbl, [hi, lo], mask=p)
                new = jnp.minimum(cur, eid)
                plsc.store_scatter(itbl, [hi, lo], new, mask=p)
                cur2 = plsc.load_gather(itbl, [hi, lo], mask=p)
                return jnp.logical_and(p, cur2 > eid)

            lax.while_loop(cond, body, sel)
            return carry2

        lax.fori_loop(0, G2, grp, 0)
        return carry

    lax.fori_loop(0, NCH2, chunk, 0)

    pltpu.sync_copy(itbl, shared.at[sid])
    plsc.subcore_barrier()
    pltpu.sync_copy(shared.at[0, pl.ds(sid * RPTR, RPTR)], mrg)
    for k in range(1, NS):
        pltpu.sync_copy(shared.at[k, pl.ds(sid * RPTR, RPTR)], tmp)

        def mn(i, c):
            mrg[i] = jnp.minimum(mrg[i], tmp[i])
            return c

        lax.fori_loop(0, RPTR, mn, 0)
    pltpu.sync_copy(mrg, out_hbm.at[cid, pl.ds(sid * RPTR, RPTR)])


def _segment_argmin(y, dst, segmax_p):
    f = pl.kernel(
        _argmin_body,
        out_type=jax.ShapeDtypeStruct((NC, NPR, 16), jnp.int32),
        scratch_types=[
            pltpu.VMEM((C2,), jnp.float32),
            pltpu.VMEM((C2,), jnp.int32),
            pltpu.VMEM((NPR, 16), jnp.float32),
            pltpu.VMEM((NPR, 16), jnp.float32),
            pltpu.VMEM((NPR, 16), jnp.int32),
            pltpu.VMEM((RPTR, 16), jnp.int32),
            pltpu.VMEM((RPTR, 16), jnp.int32),
            pltpu.VMEM_SHARED((NS, NPR, 16), jnp.int32),
            pltpu.SemaphoreType.DMA,
        ],
        **_SC_MESH,
    )
    return f(y, dst, segmax_p)


ZROWS = 64


HHALF = H // 2


def _sum_body(y2_hbm, src_hbm, dst_hbm, x_hbm, slo_hbm, shi_hbm,
              yvgA, svA, dvA, xrA, xhA, yvgB, svB, dvB, xrB, xhB,
              acc, semA, semB):
    cid = lax.axis_index("c")
    sid = lax.axis_index("s")
    wid = sid * NC + cid
    base_w = wid * EPW
    zero16 = jnp.zeros((16,), jnp.float32)

    def stage(ci, yvg, sv, dv, xrows, sem):
        base = base_w + ci * C
        pltpu.sync_copy(y2_hbm.at[pl.ds(base_w // 16 + ci * G, G)], yvg)
        pltpu.sync_copy(src_hbm.at[pl.ds(base, C)], sv)
        pltpu.sync_copy(dst_hbm.at[pl.ds(base, C)], dv)
        pltpu.async_copy(x_hbm.at[sv], xrows, sem)

    def waitg(xrows, sem):
        pltpu.make_async_copy(x_hbm.at[pl.ds(0, C)], xrows, sem).wait()

    for phase, out_hbm in ((0, slo_hbm), (1, shi_hbm)):
        hoff = phase * HHALF

        def zb(i, c):
            for j in range(HHALF // 16):
                xhA[i, pl.ds(j * 16, 16)] = zero16
            return c

        lax.fori_loop(0, C, zb, 0)
        for b in range(RPT // C):
            pltpu.sync_copy(xhA, acc.at[pl.ds(sid * RPT + b * C, C)])
        plsc.subcore_barrier()

        def process(ci, yvg, dv, xrows, xhalf, sem):
            waitg(xrows, sem)

            def rw(r, c2):
                rhi = jnp.full((16,), lax.shift_right_logical(r, 4), jnp.int32)
                rlo = jnp.full((16,), lax.bitwise_and(r, 15), jnp.int32)
                ys = plsc.load_gather(yvg, [rhi, rlo])
                for j in range(HHALF // 16):
                    xhalf[r, pl.ds(j * 16, 16)] = (
                        xrows[r, pl.ds(hoff + j * 16, 16)] * ys)
                return c2

            lax.fori_loop(0, C, rw, 0)
            pltpu.sync_copy(xhalf, acc.at[dv], add=True)

        stage(0, yvgA, svA, dvA, xrA, semA)

        def pair(pi, carry):
            ci0 = 2 * pi
            stage(ci0 + 1, yvgB, svB, dvB, xrB, semB)
            process(ci0, yvgA, dvA, xrA, xhA, semA)
            stage(ci0 + 2, yvgA, svA, dvA, xrA, semA)
            process(ci0 + 1, yvgB, dvB, xrB, xhB, semB)
            return carry

        lax.fori_loop(0, (NCHUNK - 1) // 2, pair, 0)
        process(NCHUNK - 1, yvgA, dvA, xrA, xhA, semA)
        plsc.subcore_barrier()
        pltpu.sync_copy(acc.at[pl.ds(sid * RPT, RPT)],
                        out_hbm.at[cid, pl.ds(sid * RPT, RPT)])
        plsc.subcore_barrier()


def _weighted_sum(y, src, dst, x):
    f = pl.kernel(
        _sum_body,
        out_type=[
            jax.ShapeDtypeStruct((NC, NP, HHALF), jnp.float32),
            jax.ShapeDtypeStruct((NC, NP, HHALF), jnp.float32),
        ],
        scratch_types=[
            pltpu.VMEM((G, 16), jnp.float32),
            pltpu.VMEM((C,), jnp.int32),
            pltpu.VMEM((C,), jnp.int32),
            pltpu.VMEM((C, H), jnp.float32),
            pltpu.VMEM((C, HHALF), jnp.float32),
            pltpu.VMEM((G, 16), jnp.float32),
            pltpu.VMEM((C,), jnp.int32),
            pltpu.VMEM((C,), jnp.int32),
            pltpu.VMEM((C, H), jnp.float32),
            pltpu.VMEM((C, HHALF), jnp.float32),
            pltpu.VMEM_SHARED((NP, HHALF), jnp.float32),
            pltpu.SemaphoreType.DMA,
            pltpu.SemaphoreType.DMA,
        ],
        **_SC_MESH,
    )
    return f(y.reshape(E // 16, 16), src, dst, x)


def _yhard_body(dst_hbm, id_hbm, yh_hbm, dv, yh, idtbl, itmp, sem):
    cid = lax.axis_index("c")
    sid = lax.axis_index("s")
    wid = sid * NC + cid
    base_w = wid * EPW
    iota = lax.iota(jnp.int32, 16)

    pltpu.sync_copy(id_hbm.at[0], idtbl)
    pltpu.sync_copy(id_hbm.at[1], itmp)

    def mrg_mn(i, c):
        idtbl[i] = jnp.minimum(idtbl[i], itmp[i])
        return c

    lax.fori_loop(0, NPR, mrg_mn, 0)

    def chunk(ci, carry):
        base = base_w + ci * C2
        pltpu.sync_copy(dst_hbm.at[pl.ds(base, C2)], dv)

        def grp(g, c2):
            d = dv[pl.ds(g * 16, 16)]
            eid = base + g * 16 + iota
            idg = plsc.load_gather(idtbl, [_dhi(d), _dlo(d)])
            yh[pl.ds(g * 16, 16)] = jnp.where(idg == eid, 1.0, 0.0)
            return c2

        lax.fori_loop(0, G2, grp, 0)
        pltpu.sync_copy(yh, yh_hbm.at[pl.ds(base, C2)])
        return carry

    lax.fori_loop(0, NCH2, chunk, 0)


def _hard_onehot(dst, minid_p):
    f = pl.kernel(
        _yhard_body,
        out_type=jax.ShapeDtypeStruct((E,), jnp.float32),
        scratch_types=[
            pltpu.VMEM((C2,), jnp.int32),
            pltpu.VMEM((C2,), jnp.float32),
            pltpu.VMEM((NPR, 16), jnp.int32),
            pltpu.VMEM((NPR, 16), jnp.int32),
            pltpu.SemaphoreType.DMA,
        ],
        **_SC_MESH,
    )
    return f(dst, minid_p)


def _final_body(v_ref, mask_ref, s0lo_ref, s1lo_ref, s0hi_ref, s1hi_ref, vout_ref):
    summed = jnp.concatenate(
        [s0lo_ref[0] + s1lo_ref[0], s0hi_ref[0] + s1hi_ref[0]], axis=1)
    vout_ref[...] = v_ref[...] + mask_ref[...] * summed


def _ah_body(y_ref, yh_ref, ah_ref):
    yb = y_ref[...]
    ah_ref[...] = (yh_ref[...] - yb) + yb


def _final(v, mask, sum_lo, sum_hi, y2d, yh2d):
    v_out = pl.pallas_call(
        _final_body,
        grid=(10,),
        in_specs=[
            pl.BlockSpec((ROWS_PER_BLK, H), lambda i: (i, 0)),
            pl.BlockSpec((ROWS_PER_BLK, 1), lambda i: (i, 0)),
            pl.BlockSpec((1, ROWS_PER_BLK, HHALF), lambda i: (0, i, 0)),
            pl.BlockSpec((1, ROWS_PER_BLK, HHALF), lambda i: (1, i, 0)),
            pl.BlockSpec((1, ROWS_PER_BLK, HHALF), lambda i: (0, i, 0)),
            pl.BlockSpec((1, ROWS_PER_BLK, HHALF), lambda i: (1, i, 0)),
        ],
        out_specs=pl.BlockSpec((ROWS_PER_BLK, H), lambda i: (i, 0)),
        out_shape=jax.ShapeDtypeStruct((NROW, H), jnp.float32),
    )(v, mask, sum_lo, sum_lo, sum_hi, sum_hi)
    ah2d = pl.pallas_call(
        _ah_body,
        out_shape=jax.ShapeDtypeStruct((E // H, H), jnp.float32),
    )(y2d, yh2d)
    return v_out, ah2d


def kernel(x, v, cross_edge_index, W_prog, b_prog, W_vox, b_vox, Wm1, bm1, Wm2, bm2, theta):
    src = cross_edge_index[0]
    dst = cross_edge_index[1]

    xp, vp, mask = _dense_precompute(
        x, v, W_prog, b_prog, W_vox, Wm1, bm1, Wm2, bm2)
    rot = (jnp.arange(H, dtype=jnp.int32)[:, None]
           + jnp.arange(16, dtype=jnp.int32)[None, :]) % H
    theta_b = theta.reshape(H)[rot]
    bvox_b = b_vox[rot]
    e = _edge_scores(xp, vp, src, dst, theta_b, bvox_b)

    u = jax.random.uniform(jax.random.fold_in(jax.random.key(0), 1), (E,),
                           minval=1e-10, maxval=1.0, dtype=jnp.float32)
    gumbel_noise = -jnp.log(-jnp.log(u))
    y2d = _softmax(e.reshape(E // H, H), gumbel_noise.reshape(E // H, H))
    y = y2d.reshape(E)

    segmax_p = _segment_max(y, dst)
    minid_p = _segment_argmin(y, dst, segmax_p)
    sum_lo, sum_hi = _weighted_sum(y, src, dst, x)
    yh = _hard_onehot(dst, minid_p)
    v_out, ah2d = _final(v, mask, sum_lo, sum_hi, y2d, yh.reshape(E // H, H))

    return (v_out, mask, y[:, None], ah2d.reshape(E, 1))
